# baseline (speedup 1.0000x reference)
import jax
import jax.numpy as jnp
from jax import lax
from jax.experimental import pallas as pl
from jax.experimental.pallas import tpu as pltpu
from jax.experimental.pallas import tpu_sc as plsc

B = 4096
S = 200
D = 64
NC = 2
NS = 16
NW = NC * NS
BPW = B // NW
NBUF = 4
LANES = 16


def _body(idx_hbm, table_hbm, pos_hbm, out_hbm, pos_v, idx_all, *bufs):
    rows = bufs[0:NBUF]
    gsems = bufs[NBUF:2 * NBUF]
    wsems = bufs[2 * NBUF:3 * NBUF]

    wid = lax.axis_index("s") * NC + lax.axis_index("c")
    wbase = wid * BPW
    pltpu.sync_copy(pos_hbm.at[pl.ds(0, S)], pos_v)
    pltpu.sync_copy(idx_hbm.at[pl.ds(wbase, BPW)], idx_all)

    def wait_gather(b):
        pltpu.make_async_copy(out_hbm.at[wbase], rows[b], gsems[b]).wait()

    def wait_write(b):
        pltpu.make_async_copy(rows[b], out_hbm.at[wbase], wsems[b]).wait()

    for b in range(NBUF):
        pltpu.async_copy(table_hbm.at[idx_all.at[b]], rows[b], gsems[b])

    @pl.loop(0, BPW, step=NBUF)
    def _grp(g0):
        for b in range(NBUF):
            g = g0 + b
            wait_gather(b)

            @pl.loop(0, S)
            def _row(p):
                for c in range(D // LANES):
                    sl = pl.ds(c * LANES, LANES)
                    rows[b][p, sl] += pos_v[p, sl]

            pltpu.async_copy(rows[b], out_hbm.at[wbase + g], wsems[b])

            @pl.when(g + NBUF < BPW)
            def _():
                wait_write(b)
                pltpu.async_copy(table_hbm.at[idx_all.at[g + NBUF]],
                                 rows[b], gsems[b])

    for b in range(NBUF):
        wait_write(b)


@jax.jit
def kernel(x, table, pos_encoding):
    idx = x.astype(jnp.int32)
    mesh = plsc.VectorSubcoreMesh(core_axis_name="c", subcore_axis_name="s")
    return pl.kernel(
        _body,
        out_type=jax.ShapeDtypeStruct((B, S, D), jnp.float32),
        mesh=mesh,
        compiler_params=pltpu.CompilerParams(use_tc_tiling_on_sc=False),
        scratch_types=[
            pltpu.VMEM((S, D), jnp.float32),
            pltpu.VMEM((BPW, S), jnp.int32),
        ] + [pltpu.VMEM((S, D), jnp.float32) for _ in range(NBUF)]
          + [pltpu.SemaphoreType.DMA for _ in range(2 * NBUF)],
    )(idx, table, pos_encoding)
